# Initial kernel scaffold; baseline (speedup 1.0000x reference)
import jax
import jax.numpy as jnp
from jax.experimental import pallas as pl


def kernel(feats, edge_index, edge_index_aug, q, W_q1, b_q1, W_l1, b_l1, W_f, b_f, q_att, n_att, fq_att, f_att, W_lq, b_lq, W_lf, b_lf):
    raise NotImplementedError("write your pallas kernel here")


# speedup vs baseline: 12.4219x; 12.4219x over previous
import functools

import jax
import jax.numpy as jnp
from jax import lax
from jax.experimental import pallas as pl
from jax.experimental.pallas import tpu as pltpu
from jax.experimental.pallas import tpu_sc as plsc

_NS = 16


def _pad_n(n):
    per = -(-n // _NS)
    per = -(-per // 40) * 40
    return per * _NS
_C = 80
_B = 1000


def _make_counts_kernel(n, e):
    per_tile = e // _NS
    steps = per_tile // _C
    np_ = _pad_n(n)
    assert np_ >= n + 8
    rpt = np_ // _NS
    zrows = rpt // 5
    trash = np_ - 8
    mesh = plsc.VectorSubcoreMesh(core_axis_name="c", subcore_axis_name="s")

    @functools.partial(
        pl.kernel,
        out_type=jax.ShapeDtypeStruct((4 * np_, 16), jnp.float32),
        mesh=mesh,
        scratch_types=[
            pltpu.VMEM_SHARED((np_, 16), jnp.float32),
            pltpu.VMEM_SHARED((np_, 16), jnp.float32),
            pltpu.VMEM((_C, 16), jnp.float32),
            pltpu.VMEM((zrows, 16), jnp.float32),
            pltpu.VMEM((_C,), jnp.int32),
            pltpu.VMEM((_C,), jnp.int32),
            pltpu.VMEM((_C,), jnp.int32),
            pltpu.VMEM((16,), jnp.int32),
        ],
    )
    def counts_kernel(src, dst, src_a, dst_a, qv, out, acc, acc2, block,
                      zbuf, sidx, didx, didx2, qvm):
        c = lax.axis_index("c")
        s = lax.axis_index("s")
        pltpu.sync_copy(qv, qvm)
        zeros16 = jnp.zeros((16,), jnp.float32)
        one0 = jnp.where(lax.iota(jnp.int32, 16) == 0, 1.0, 0.0)

        def zrow(r, _):
            zbuf[r, :] = zeros16
            return 0
        lax.fori_loop(0, zrows, zrow, 0)

        def brow(r, _):
            block[r, :] = one0
            return 0
        lax.fori_loop(0, _C, brow, 0)

        row0 = s * rpt
        for j in range(rpt // zrows):
            pltpu.sync_copy(zbuf, acc.at[pl.ds(row0 + j * zrows, zrows)])
            pltpu.sync_copy(zbuf, acc2.at[pl.ds(row0 + j * zrows, zrows)])
        plsc.subcore_barrier()

        qvec = qvm[...]
        base = s * per_tile

        def run(esrc, edst):
            def step(g, _):
                off = base + g * _C
                pltpu.sync_copy(esrc.at[pl.ds(off, _C)], sidx)
                pltpu.sync_copy(edst.at[pl.ds(off, _C)], didx)
                for j8 in range(_C // 16):
                    sv = sidx[pl.ds(j8 * 16, 16)]
                    dv = didx[pl.ds(j8 * 16, 16)]
                    didx2[pl.ds(j8 * 16, 16)] = jnp.where(sv == qvec, dv,
                                                          trash)
                pltpu.sync_copy(block, acc.at[didx], add=True)
                pltpu.sync_copy(block, acc2.at[didx2], add=True)
                return 0
            lax.fori_loop(0, steps, step, 0)

        @pl.when(c == 0)
        def _():
            run(src, dst)

        @pl.when(c == 1)
        def _():
            run(src_a, dst_a)

        plsc.subcore_barrier()
        pltpu.sync_copy(acc.at[pl.ds(row0, rpt)],
                        out.at[pl.ds(c * np_ + row0, rpt)])
        pltpu.sync_copy(acc2.at[pl.ds(row0, rpt)],
                        out.at[pl.ds(2 * np_ + c * np_ + row0, rpt)])

    return counts_kernel


def _make_prop_kernel(n, e):
    per_tile_full = e // _NS
    steps_full = per_tile_full // _C
    per_tile_half = e // (2 * _NS)
    steps_half = per_tile_half // _C
    np_ = _pad_n(n)
    rpt = np_ // _NS
    zrows = rpt // 5
    mesh = plsc.VectorSubcoreMesh(core_axis_name="c", subcore_axis_name="s")

    @functools.partial(
        pl.kernel,
        out_type=(
            jax.ShapeDtypeStruct((np_, 128), jnp.float32),
            jax.ShapeDtypeStruct((np_, 128), jnp.float32),
            jax.ShapeDtypeStruct((2 * np_, 128), jnp.float32),
        ),
        mesh=mesh,
        scratch_types=[
            pltpu.VMEM_SHARED((np_, 128), jnp.float32),
            pltpu.VMEM((zrows, 128), jnp.float32),
            pltpu.VMEM((_C, 128), jnp.float32),
            pltpu.VMEM((_C,), jnp.int32),
            pltpu.VMEM((_C,), jnp.int32),
            pltpu.SemaphoreType.DMA,
        ],
    )
    def prop_kernel(y1, y1a, yf, src, dst, src_a, dst_a, o1, o1a, ofp,
                    acc, zbuf, rows, sidx, didx, sem):
        c = lax.axis_index("c")
        s = lax.axis_index("s")
        zeros16 = jnp.zeros((16,), jnp.float32)

        def zrow(r, _):
            for k in range(8):
                zbuf[r, pl.ds(k * 16, 16)] = zeros16
            return 0
        lax.fori_loop(0, zrows, zrow, 0)

        row0 = s * rpt

        def zero_acc():
            for j in range(rpt // zrows):
                pltpu.sync_copy(zbuf, acc.at[pl.ds(row0 + j * zrows, zrows)])

        def run_edges(table, esrc, edst, base, steps):
            def step(g, _):
                off = base + g * _C
                pltpu.sync_copy(esrc.at[pl.ds(off, _C)], sidx)
                pltpu.sync_copy(edst.at[pl.ds(off, _C)], didx)
                pltpu.async_copy(table.at[sidx], rows, sem).wait()
                pltpu.sync_copy(rows, acc.at[didx], add=True)
                return 0
            lax.fori_loop(0, steps, step, 0)

        zero_acc()
        plsc.subcore_barrier()

        @pl.when(c == 0)
        def _():
            run_edges(y1, src, dst, s * per_tile_full, steps_full)

        @pl.when(c == 1)
        def _():
            run_edges(y1a, src_a, dst_a, s * per_tile_full, steps_full)

        plsc.subcore_barrier()

        @pl.when(c == 0)
        def _():
            pltpu.sync_copy(acc.at[pl.ds(row0, rpt)], o1.at[pl.ds(row0, rpt)])

        @pl.when(c == 1)
        def _():
            pltpu.sync_copy(acc.at[pl.ds(row0, rpt)],
                            o1a.at[pl.ds(row0, rpt)])

        plsc.subcore_barrier()

        zero_acc()
        plsc.subcore_barrier()
        run_edges(yf, src, dst, c * (e // 2) + s * per_tile_half, steps_half)
        plsc.subcore_barrier()
        pltpu.sync_copy(acc.at[pl.ds(row0, rpt)],
                        ofp.at[pl.ds(c * np_ + row0, rpt)])

    return prop_kernel


def _s2_body(qi_ref, fscal_ref, feats_ref, deg_ref, dega_ref, kq_ref,
             kqa_ref, wl1_ref,
             wlf_ref, blf_ref, wlq_ref, blq_ref, fqatt_ref, fatt_ref,
             wq1_ref, bq1_ref, wf_ref,
             y1_ref, y1a_ref, yf_ref, hq_ref, hqa_ref, hfmid_ref):
    i = pl.program_id(0)
    feats = feats_ref[...]
    dinv = lax.rsqrt(deg_ref[:, 0:1] + 1.0)
    dinva = lax.rsqrt(dega_ref[:, 0:1] + 1.0)
    kq = kq_ref[:, 0:1]
    kqa = kqa_ref[:, 0:1]
    xw1 = jnp.dot(feats, wl1_ref[...], preferred_element_type=jnp.float32)
    y1_ref[...] = dinv * xw1
    y1a_ref[...] = dinva * xw1
    qs = qi_ref[0]
    dinv_q = fscal_ref[0]
    dinv_aq = fscal_ref[1]
    rowid = i * _B + lax.broadcasted_iota(jnp.int32, (_B, 1), 0)
    ismask = (rowid == qs).astype(jnp.float32)
    coef = dinv * (dinv_q * kq) + ismask * (dinv_q * dinv_q)
    coefa = dinva * (dinv_aq * kqa) + ismask * (dinv_aq * dinv_aq)
    hq_ref[...] = jnp.maximum(coef * wq1_ref[...] + bq1_ref[...], 0.0)
    hqa_ref[...] = jnp.maximum(coefa * wq1_ref[...] + bq1_ref[...], 0.0)
    f2 = jnp.dot(feats, wlf_ref[...],
                 preferred_element_type=jnp.float32) + blf_ref[...]
    q2 = blq_ref[...] + ismask * wlq_ref[...]
    u = jnp.sum(q2 * fqatt_ref[...], axis=1, keepdims=True)
    v = jnp.sum(f2 * fatt_ref[...], axis=1, keepdims=True)
    m = jnp.maximum(u, v)
    eu = jnp.exp(u - m)
    ev = jnp.exp(v - m)
    hfmid = (eu * q2 + ev * f2) / (eu + ev)
    hfmid_ref[...] = hfmid
    yf_ref[...] = dinv * jnp.dot(hfmid, wf_ref[...],
                                 preferred_element_type=jnp.float32)


def _s4_body(acc1_ref, acc1a_ref, of0_ref, of1_ref, y1_ref, y1a_ref, yf_ref,
             hq_ref, hqa_ref, hfmid_ref, deg_ref, dega_ref, bl1_ref, bf_ref,
             qatt_ref, natt_ref, hf_ref, haug_ref):
    dinv = lax.rsqrt(deg_ref[:, 0:1] + 1.0)
    dinva = lax.rsqrt(dega_ref[:, 0:1] + 1.0)
    bl1 = bl1_ref[...]
    qatt = qatt_ref[...]
    natt = natt_ref[...]
    h = jnp.maximum(dinv * (acc1_ref[...] + y1_ref[...]) + bl1, 0.0)
    ha = jnp.maximum(dinva * (acc1a_ref[...] + y1a_ref[...]) + bl1, 0.0)

    def fuse(a, b):
        u = jnp.sum(a * qatt, axis=1, keepdims=True)
        v = jnp.sum(b * natt, axis=1, keepdims=True)
        m = jnp.maximum(u, v)
        eu = jnp.exp(u - m)
        ev = jnp.exp(v - m)
        return (eu * a + ev * b) / (eu + ev)

    hq = hq_ref[...]
    hqa = hqa_ref[...]
    hf_fused = fuse(hq, h)
    haug_ref[...] = fuse(hqa, ha)
    gcnf = dinv * (of0_ref[...] + of1_ref[...] + yf_ref[...]) + bf_ref[...]
    hf_ref[...] = jnp.maximum(hf_fused + gcnf, 0.0)


def _row_spec(bs=None):
    if bs is None:
        bs = (_B, 128)
    return pl.BlockSpec(bs, lambda i: (i, 0))


def _full_spec(shape):
    return pl.BlockSpec(shape, lambda i: tuple(0 for _ in shape))


def kernel(feats, edge_index, edge_index_aug, q, W_q1, b_q1, W_l1, b_l1,
           W_f, b_f, q_att, n_att, fq_att, f_att, W_lq, b_lq, W_lf, b_lf):
    n, d = feats.shape
    e = edge_index.shape[1]
    h = W_l1.shape[1]
    assert e % (2 * _NS * _C) == 0 and h == 128
    np_ = _pad_n(n)

    src = edge_index[0]
    dst = edge_index[1]
    src_a = edge_index_aug[0]
    dst_a = edge_index_aug[1]
    qs = jnp.asarray(q, jnp.int32)
    qv = jnp.full((16,), qs, jnp.int32)

    counts_p = _make_counts_kernel(n, e)(src, dst, src_a, dst_a, qv)
    deg_ei = counts_p[0:n]
    deg_aug = counts_p[np_:np_ + n]
    kq_ei = counts_p[2 * np_:2 * np_ + n]
    kq_aug = counts_p[3 * np_:3 * np_ + n]

    dinv_q = lax.rsqrt(counts_p[qs, 0] + 1.0)
    dinv_aq = lax.rsqrt(counts_p[np_ + qs, 0] + 1.0)
    fscal = jnp.stack([dinv_q, dinv_aq])
    qi = qs.reshape(1)

    row128 = lambda a: a.reshape(1, h)
    grid = (n // _B,)
    obl = jax.ShapeDtypeStruct((n, h), jnp.float32)
    smem_spec = pl.BlockSpec(memory_space=pltpu.SMEM)

    y1, y1a, yf, hq, hqa, hfmid = pl.pallas_call(
        _s2_body,
        grid=grid,
        in_specs=[smem_spec, smem_spec, _row_spec()] +
                 [_row_spec((_B, 16))] * 4 +
                 [_full_spec((h, h)), _full_spec((h, h)),
                  _full_spec((1, h)), _full_spec((1, h)), _full_spec((1, h)),
                  _full_spec((1, h)), _full_spec((1, h)), _full_spec((1, h)),
                  _full_spec((1, h)), _full_spec((h, h))],
        out_specs=[_row_spec()] * 6,
        out_shape=[obl] * 6,
    )(qi, fscal, feats, deg_ei, deg_aug, kq_ei, kq_aug, W_l1, W_lf,
      row128(b_lf), W_lq, row128(b_lq), fq_att.reshape(1, h),
      f_att.reshape(1, h), W_q1, row128(b_q1), W_f)

    y1p = jnp.zeros((np_, h), jnp.float32).at[:n].set(y1)
    y1ap = jnp.zeros((np_, h), jnp.float32).at[:n].set(y1a)
    yfp = jnp.zeros((np_, h), jnp.float32).at[:n].set(yf)
    o1p, o1ap, ofpp = _make_prop_kernel(n, e)(y1p, y1ap, yfp, src, dst,
                                              src_a, dst_a)
    o1, o1a = o1p[:n], o1ap[:n]
    ofp = jnp.concatenate([ofpp[:n], ofpp[np_:np_ + n]], axis=0)

    hf, haug = pl.pallas_call(
        _s4_body,
        grid=grid,
        in_specs=[_row_spec()] * 10 + [_row_spec((_B, 16))] * 2 +
                 [_full_spec((1, h))] * 4,
        out_specs=[_row_spec()] * 2,
        out_shape=[obl] * 2,
    )(o1, o1a, ofp[:n], ofp[n:], y1, y1a, yf, hq, hqa, hfmid,
      deg_ei, deg_aug, row128(b_l1), row128(b_f),
      q_att.reshape(1, h), n_att.reshape(1, h))

    return (hf, haug)
